# baseline (speedup 1.0000x reference)
import jax
import jax.numpy as jnp
from jax import lax
from jax.experimental import pallas as pl
from jax.experimental.pallas import tpu as pltpu
from jax.experimental.pallas import tpu_sc as plsc

N_NODES = 10000
N_EDGES = 320000
NODEDIM = 128
EDGEDIM = 16

NW = 32
CHUNK = 128
NCHUNKS = N_EDGES // CHUNK
CPW = 88
NCHUNKS_PAD = 2504


def _tables_body(nodes8_ref, wr8_ref, ws8_ref, tr_ref, ts_ref):
    n = nodes8_ref[...]
    tr_ref[...] = jnp.dot(n, wr8_ref[...], preferred_element_type=jnp.float32)
    ts_ref[...] = jnp.dot(n, ws8_ref[...], preferred_element_type=jnp.float32)


BG = 125


def _base_body(edges_t_ref, wet_ref, bcol_ref, out_hbm, scratch, sem):
    i = pl.program_id(0)
    bt = (
        jnp.dot(wet_ref[...], edges_t_ref[...], preferred_element_type=jnp.float32)
        + bcol_ref[...]
    )
    for g in range(BG):
        scratch[g] = bt[:, g * CHUNK:(g + 1) * CHUNK]
    pltpu.async_copy(scratch, out_hbm.at[pl.ds(i * BG, BG)], sem).wait()


BROWS = CHUNK // 8


EPW = CPW * CHUNK
FLAT = CHUNK * EDGEDIM
PC = 2


def _sc_body(tr_hbm, ts_hbm, ridx_hbm, sidx_hbm, base_hbm, out_hbm,
             ridx_v, sidx_v, buf_r, buf_s, buf_b, buf_w, sem_in, sem_st):
    cid = lax.axis_index("c")
    sid = lax.axis_index("s")
    wid = sid * 2 + cid

    def _start_of(w):
        return ((w * NCHUNKS) // NW) & ~7

    start = pl.multiple_of(_start_of(wid), 8)
    nxt = jnp.where(wid == NW - 1, NCHUNKS, _start_of(wid + 1))
    cnt = nxt - start
    e0 = pl.multiple_of(jnp.minimum(start * CHUNK, N_EDGES - EPW), CHUNK)
    off = start * CHUNK - e0
    pltpu.sync_copy(ridx_hbm.at[pl.ds(e0, EPW)], ridx_v)
    pltpu.sync_copy(sidx_hbm.at[pl.ds(e0, EPW)], sidx_v)

    def issue_in(g, k):
        le = pl.multiple_of(off + g * CHUNK, CHUNK)
        c = start + g
        for q in range(PC):
            pltpu.async_copy(
                tr_hbm.at[ridx_v.at[pl.ds(le + q * CHUNK, CHUNK)]],
                buf_r[k].at[pl.ds(q * CHUNK, CHUNK)], sem_in[k])
            pltpu.async_copy(
                ts_hbm.at[sidx_v.at[pl.ds(le + q * CHUNK, CHUNK)]],
                buf_s[k].at[pl.ds(q * CHUNK, CHUNK)], sem_in[k])
        pltpu.async_copy(base_hbm.at[pl.ds(c, PC)], buf_b[k], sem_in[k])

    def wait_in(k):
        for q in range(PC):
            pltpu.make_async_copy(
                tr_hbm.at[pl.ds(0, CHUNK)],
                buf_r[k].at[pl.ds(0, CHUNK)], sem_in[k]).wait()
            pltpu.make_async_copy(
                ts_hbm.at[pl.ds(0, CHUNK)],
                buf_s[k].at[pl.ds(0, CHUNK)], sem_in[k]).wait()
        pltpu.make_async_copy(base_hbm.at[pl.ds(0, PC)], buf_b[k], sem_in[k]).wait()

    def wait_st(k):
        pltpu.make_async_copy(
            buf_w[k].at[0], out_hbm.at[0, pl.ds(0, PC)], sem_st[k]).wait()
        pltpu.make_async_copy(
            buf_w[k].at[1], out_hbm.at[1, pl.ds(0, PC)], sem_st[k]).wait()

    iota16 = lax.iota(jnp.int32, 16)
    zero16 = iota16 & 0

    def phase(g, k, p):
        wait_in(k)

        @pl.when(p >= 1)
        def _():
            wait_st(k)

        for q in range(PC):

            @plsc.parallel_loop(0, EDGEDIM * 8, unroll=16)
            def _(i):
                f = i >> 3
                eg = i & 7
                rows = iota16 + (eg * 16 + q * CHUNK)
                cols = zero16 + f
                vr = plsc.load_gather(buf_r[k], [rows, cols])
                vs = plsc.load_gather(buf_s[k], [rows, cols])
                buf_w[k][f >> 3, q, f & 7, pl.ds(eg * 16, 16)] = (
                    buf_b[k][q, f, pl.ds(eg * 16, 16)] + vr + vs
                )

        c = start + g
        pltpu.async_copy(buf_w[k].at[0], out_hbm.at[0, pl.ds(c, PC)], sem_st[k])
        pltpu.async_copy(buf_w[k].at[1], out_hbm.at[1, pl.ds(c, PC)], sem_st[k])

        @pl.when(g + 2 * PC < cnt)
        def _():
            issue_in(g + 2 * PC, k)

    issue_in(0, 0)
    issue_in(PC, 1)

    def pair_body(p, carry):
        phase(2 * PC * p, 0, p)
        phase(2 * PC * p + PC, 1, p)
        return carry

    lax.fori_loop(0, cnt // (2 * PC), pair_body, 0)
    wait_st(0)
    wait_st(1)


_sc_gather_add = pl.kernel(
    _sc_body,
    out_type=jax.ShapeDtypeStruct((2, NCHUNKS, 8, 128), jnp.float32),
    mesh=plsc.VectorSubcoreMesh(core_axis_name="c", subcore_axis_name="s"),
    scratch_types=[
        pltpu.VMEM((EPW,), jnp.int32),
        pltpu.VMEM((EPW,), jnp.int32),
        [pltpu.VMEM((PC * CHUNK, EDGEDIM), jnp.float32) for _ in range(2)],
        [pltpu.VMEM((PC * CHUNK, EDGEDIM), jnp.float32) for _ in range(2)],
        [pltpu.VMEM((PC, EDGEDIM, CHUNK), jnp.float32) for _ in range(2)],
        [pltpu.VMEM((2, PC, 8, CHUNK), jnp.float32) for _ in range(2)],
        [pltpu.SemaphoreType.DMA for _ in range(2)],
        [pltpu.SemaphoreType.DMA for _ in range(2)],
    ],
    compiler_params=pltpu.CompilerParams(
        use_tc_tiling_on_sc=False, needs_layout_passes=False),
)


def kernel(nodes, edges, senders, receivers, W, b):
    f32 = jnp.float32
    w_e = W[:EDGEDIM]
    w_r = W[EDGEDIM:EDGEDIM + NODEDIM]
    w_s = W[EDGEDIM + NODEDIM:]

    eye8 = jnp.eye(8, dtype=f32)
    wr8 = jnp.kron(eye8, w_r)
    ws8 = jnp.kron(eye8, w_s)
    nodes8 = nodes.reshape(N_NODES // 8, 8 * NODEDIM)
    tr8, ts8 = pl.pallas_call(
        _tables_body,
        out_shape=(
            jax.ShapeDtypeStruct((N_NODES // 8, 128), f32),
            jax.ShapeDtypeStruct((N_NODES // 8, 128), f32),
        ),
    )(nodes8, wr8, ws8)
    table_r = tr8.reshape(N_NODES, EDGEDIM)
    table_s = ts8.reshape(N_NODES, EDGEDIM)

    edges_t = edges.T
    w_et = w_e.T
    b_col = b.reshape(EDGEDIM, 1)
    base_sc = pl.pallas_call(
        _base_body,
        grid=(NCHUNKS // BG,),
        in_specs=[
            pl.BlockSpec((EDGEDIM, BG * CHUNK), lambda i: (0, i)),
            pl.BlockSpec((EDGEDIM, EDGEDIM), lambda i: (0, 0)),
            pl.BlockSpec((EDGEDIM, 1), lambda i: (0, 0)),
        ],
        out_specs=pl.BlockSpec(memory_space=pltpu.MemorySpace.HBM),
        out_shape=jax.ShapeDtypeStruct((NCHUNKS, EDGEDIM, CHUNK), f32),
        scratch_shapes=[
            pltpu.VMEM((BG, EDGEDIM, CHUNK), f32),
            pltpu.SemaphoreType.DMA,
        ],
    )(edges_t, w_et, b_col)

    out4 = _sc_gather_add(table_r, table_s, receivers, senders, base_sc)
    return out4.transpose(1, 3, 0, 2).reshape(N_EDGES, EDGEDIM)


# speedup vs baseline: 1.0647x; 1.0647x over previous
import jax
import jax.numpy as jnp
from jax import lax
from jax.experimental import pallas as pl
from jax.experimental.pallas import tpu as pltpu
from jax.experimental.pallas import tpu_sc as plsc

N_NODES = 10000
N_EDGES = 320000
NODEDIM = 128
EDGEDIM = 16

NW = 32
CHUNK = 128
NCHUNKS = N_EDGES // CHUNK
CPW = 88
NCHUNKS_PAD = 2504


def _tables_body(nodes8_ref, wr8_ref, ws8_ref, tr_ref, ts_ref):
    n = nodes8_ref[...]
    tr_ref[...] = jnp.dot(n, wr8_ref[...], preferred_element_type=jnp.float32)
    ts_ref[...] = jnp.dot(n, ws8_ref[...], preferred_element_type=jnp.float32)


BG = 125


def _base_body(edges_t_ref, wet_ref, bcol_ref, out_ref):
    bt = (
        jnp.dot(wet_ref[...], edges_t_ref[...], preferred_element_type=jnp.float32)
        + bcol_ref[...]
    )
    for g in range(BG):
        out_ref[g] = bt[:, g * CHUNK:(g + 1) * CHUNK]


BROWS = CHUNK // 8


EPW = CPW * CHUNK
FLAT = CHUNK * EDGEDIM
PC = 2


def _sc_body(tr_hbm, ts_hbm, ridx_hbm, sidx_hbm, base_hbm, out_hbm,
             ridx_v, sidx_v, buf_r, buf_b, buf_w, sem_r, sem_s, sem_st):
    cid = lax.axis_index("c")
    sid = lax.axis_index("s")
    wid = sid * 2 + cid

    def _start_of(w):
        return ((w * NCHUNKS) // NW) & ~7

    start = pl.multiple_of(_start_of(wid), 8)
    nxt = jnp.where(wid == NW - 1, NCHUNKS, _start_of(wid + 1))
    cnt = nxt - start
    e0 = pl.multiple_of(jnp.minimum(start * CHUNK, N_EDGES - EPW), CHUNK)
    off = start * CHUNK - e0
    pltpu.sync_copy(ridx_hbm.at[pl.ds(e0, EPW)], ridx_v)
    pltpu.sync_copy(sidx_hbm.at[pl.ds(e0, EPW)], sidx_v)

    def issue_r(g, k):
        le = pl.multiple_of(off + g * CHUNK, CHUNK)
        c = start + g
        for q in range(PC):
            pltpu.async_copy(
                tr_hbm.at[ridx_v.at[pl.ds(le + q * CHUNK, CHUNK)]],
                buf_r[k].at[pl.ds(q * CHUNK, CHUNK)], sem_r[k])
        pltpu.async_copy(base_hbm.at[pl.ds(c, PC)], buf_b[k], sem_r[k])

    def issue_s(g, k):
        le = pl.multiple_of(off + g * CHUNK, CHUNK)
        for q in range(PC):
            pltpu.async_copy(
                ts_hbm.at[sidx_v.at[pl.ds(le + q * CHUNK, CHUNK)]],
                buf_r[k].at[pl.ds(q * CHUNK, CHUNK)], sem_s[k], add=True)

    def wait_r(k):
        for q in range(PC):
            pltpu.make_async_copy(
                tr_hbm.at[pl.ds(0, CHUNK)],
                buf_r[k].at[pl.ds(0, CHUNK)], sem_r[k]).wait()
        pltpu.make_async_copy(base_hbm.at[pl.ds(0, PC)], buf_b[k], sem_r[k]).wait()

    def wait_s(k):
        for q in range(PC):
            pltpu.make_async_copy(
                ts_hbm.at[pl.ds(0, CHUNK)],
                buf_r[k].at[pl.ds(0, CHUNK)], sem_s[k]).wait()

    def wait_st(k):
        pltpu.make_async_copy(
            buf_w[k].at[0], out_hbm.at[0, pl.ds(0, PC)], sem_st[k]).wait()
        pltpu.make_async_copy(
            buf_w[k].at[1], out_hbm.at[1, pl.ds(0, PC)], sem_st[k]).wait()

    iota16 = lax.iota(jnp.int32, 16)
    zero16 = iota16 & 0

    def phase(g, k, k2, p):
        wait_s(k)

        @pl.when(p >= 1)
        def _():
            wait_st(k)

        for q in range(PC):

            @plsc.parallel_loop(0, EDGEDIM * 8, unroll=16)
            def _(i):
                f = i >> 3
                eg = i & 7
                rows = iota16 + (eg * 16 + q * CHUNK)
                cols = zero16 + f
                vr = plsc.load_gather(buf_r[k], [rows, cols])
                buf_w[k][f >> 3, q, f & 7, pl.ds(eg * 16, 16)] = (
                    buf_b[k][q, f, pl.ds(eg * 16, 16)] + vr
                )

        c = start + g
        pltpu.async_copy(buf_w[k].at[0], out_hbm.at[0, pl.ds(c, PC)], sem_st[k])
        pltpu.async_copy(buf_w[k].at[1], out_hbm.at[1, pl.ds(c, PC)], sem_st[k])

        @pl.when(g + PC < cnt)
        def _():
            wait_r(k2)
            issue_s(g + PC, k2)

        @pl.when(g + 2 * PC < cnt)
        def _():
            issue_r(g + 2 * PC, k)

    issue_r(0, 0)
    issue_r(PC, 1)
    wait_r(0)
    issue_s(0, 0)

    def pair_body(p, carry):
        phase(2 * PC * p, 0, 1, p)
        phase(2 * PC * p + PC, 1, 0, p)
        return carry

    lax.fori_loop(0, cnt // (2 * PC), pair_body, 0)
    wait_st(0)
    wait_st(1)


_sc_gather_add = pl.kernel(
    _sc_body,
    out_type=jax.ShapeDtypeStruct((2, NCHUNKS, 8, 128), jnp.float32),
    mesh=plsc.VectorSubcoreMesh(core_axis_name="c", subcore_axis_name="s"),
    scratch_types=[
        pltpu.VMEM((EPW,), jnp.int32),
        pltpu.VMEM((EPW,), jnp.int32),
        [pltpu.VMEM((PC * CHUNK, EDGEDIM), jnp.float32) for _ in range(2)],
        [pltpu.VMEM((PC, EDGEDIM, CHUNK), jnp.float32) for _ in range(2)],
        [pltpu.VMEM((2, PC, 8, CHUNK), jnp.float32) for _ in range(2)],
        [pltpu.SemaphoreType.DMA for _ in range(2)],
        [pltpu.SemaphoreType.DMA for _ in range(2)],
        [pltpu.SemaphoreType.DMA for _ in range(2)],
    ],
    compiler_params=pltpu.CompilerParams(
        use_tc_tiling_on_sc=False, needs_layout_passes=False),
)


def kernel(nodes, edges, senders, receivers, W, b):
    f32 = jnp.float32
    w_e = W[:EDGEDIM]
    w_r = W[EDGEDIM:EDGEDIM + NODEDIM]
    w_s = W[EDGEDIM + NODEDIM:]

    eye8 = jnp.eye(8, dtype=f32)
    wr8 = jnp.kron(eye8, w_r)
    ws8 = jnp.kron(eye8, w_s)
    nodes8 = nodes.reshape(N_NODES // 8, 8 * NODEDIM)
    tr8, ts8 = pl.pallas_call(
        _tables_body,
        out_shape=(
            jax.ShapeDtypeStruct((N_NODES // 8, 128), f32),
            jax.ShapeDtypeStruct((N_NODES // 8, 128), f32),
        ),
    )(nodes8, wr8, ws8)
    table_r = tr8.reshape(N_NODES, EDGEDIM)
    table_s = ts8.reshape(N_NODES, EDGEDIM)

    edges_t = edges.T
    w_et = w_e.T
    b_col = b.reshape(EDGEDIM, 1)
    base_sc = pl.pallas_call(
        _base_body,
        grid=(NCHUNKS // BG,),
        in_specs=[
            pl.BlockSpec((EDGEDIM, BG * CHUNK), lambda i: (0, i)),
            pl.BlockSpec((EDGEDIM, EDGEDIM), lambda i: (0, 0)),
            pl.BlockSpec((EDGEDIM, 1), lambda i: (0, 0)),
        ],
        out_specs=pl.BlockSpec((BG, EDGEDIM, CHUNK), lambda i: (i, 0, 0)),
        out_shape=jax.ShapeDtypeStruct((NCHUNKS, EDGEDIM, CHUNK), f32),
    )(edges_t, w_et, b_col)

    out4 = _sc_gather_add(table_r, table_s, receivers, senders, base_sc)
    return out4.transpose(1, 3, 0, 2).reshape(N_EDGES, EDGEDIM)
